# Initial kernel scaffold; baseline (speedup 1.0000x reference)
import jax
import jax.numpy as jnp
from jax.experimental import pallas as pl


def kernel(node_feat, edge_index, W1, b1, s1, W2, b2, s2, cls, bias_dec):
    raise NotImplementedError("write your pallas kernel here")


# speedup vs baseline: 5.5794x; 5.5794x over previous
import functools

import jax
import jax.numpy as jnp
from jax import lax
from jax.experimental import pallas as pl
from jax.experimental.pallas import tpu as pltpu
from jax.experimental.pallas import tpu_sc as plsc

_N = 10000
_E = 320000
_DH = 129
_P = 144
_EB = 128
_NC = 2
_NS = 16
_NW = _NC * _NS
_STEPS = _E // _EB
_SPW = -(-_STEPS // _NW)
_NA = 10240
_RPS = _NA // _NS
_BN = 1000


def _lorentz_normalize(y, s_raw):
    sfac = jnp.minimum(jnp.exp(s_raw), 10.0)
    t = sfac / (1.0 + jnp.exp(-y[:, 0:1])) + 1.5
    sq = jnp.sum(y * y, axis=1, keepdims=True) - y[:, 0:1] * y[:, 0:1]
    sq = jnp.maximum(sq, 1e-8)
    fac = jnp.sqrt(jnp.maximum((t * t - 1.0) / sq, 1e-8))
    col = lax.broadcasted_iota(jnp.int32, y.shape, 1)
    return jnp.where(col == 0, t, y * fac)


def _expmap_linear_body(x_ref, w_ref, wt_ref, b_ref, s_ref, o_ref):
    x = x_ref[...]
    nrm = jnp.maximum(jnp.sqrt(jnp.sum(x * x, axis=1, keepdims=True)), 1e-8)
    e = jnp.exp(nrm)
    ei = 1.0 / e
    time = 0.5 * (e + ei)
    coef = (0.5 * (e - ei)) / nrm
    y = jnp.dot(coef * x, w_ref[...], preferred_element_type=jnp.float32)
    y = y + time * wt_ref[...] + b_ref[...]
    o_ref[...] = _lorentz_normalize(y, s_ref[0, 0])


def _agg_linear_body(p0_ref, p1_ref, w_ref, b_ref, s_ref, o_ref):
    sup = p0_ref[0] + p1_ref[0]
    t0 = sup[:, 0:1]
    inner = jnp.sum(sup * sup, axis=1, keepdims=True) - 2.0 * t0 * t0
    denom = jnp.sqrt(jnp.maximum(jnp.abs(inner), 1e-8))
    xr = jnp.maximum(sup / denom, 0.0)
    y = jnp.dot(xr, w_ref[...], preferred_element_type=jnp.float32) + b_ref[...]
    o_ref[...] = _lorentz_normalize(y, s_ref[0, 0])


def _agg_decode_body(p0_ref, p1_ref, cls_ref, b_ref, o_ref):
    sup = p0_ref[0] + p1_ref[0]
    t0 = sup[:, 0:1]
    inner = jnp.sum(sup * sup, axis=1, keepdims=True) - 2.0 * t0 * t0
    denom = jnp.sqrt(jnp.maximum(jnp.abs(inner), 1e-8))
    h = sup / denom
    col = lax.broadcasted_iota(jnp.int32, h.shape, 1)
    xm = jnp.where(col == 0, -h, h)
    o_ref[...] = (2.0 + 2.0 * jnp.dot(xm, cls_ref[...],
                                      preferred_element_type=jnp.float32)
                  + b_ref[...])


def _tc_expmap_linear(x, w, wt, b, s):
    return pl.pallas_call(
        _expmap_linear_body,
        grid=(_N // _BN,),
        in_specs=[
            pl.BlockSpec((_BN, 128), lambda i: (i, 0)),
            pl.BlockSpec((128, _P), lambda i: (0, 0)),
            pl.BlockSpec((1, _P), lambda i: (0, 0)),
            pl.BlockSpec((1, _P), lambda i: (0, 0)),
            pl.BlockSpec((1, 1), lambda i: (0, 0)),
        ],
        out_specs=pl.BlockSpec((_BN, _P), lambda i: (i, 0)),
        out_shape=jax.ShapeDtypeStruct((_N, _P), jnp.float32),
    )(x, w, wt, b, s)


def _tc_agg_linear(parts, w, b, s):
    return pl.pallas_call(
        _agg_linear_body,
        grid=(_N // _BN,),
        in_specs=[
            pl.BlockSpec((1, _BN, _P), lambda i: (0, i, 0)),
            pl.BlockSpec((1, _BN, _P), lambda i: (1, i, 0)),
            pl.BlockSpec((_P, _P), lambda i: (0, 0)),
            pl.BlockSpec((1, _P), lambda i: (0, 0)),
            pl.BlockSpec((1, 1), lambda i: (0, 0)),
        ],
        out_specs=pl.BlockSpec((_BN, _P), lambda i: (i, 0)),
        out_shape=jax.ShapeDtypeStruct((_N, _P), jnp.float32),
    )(parts, parts, w, b, s)


def _tc_agg_decode(parts, clsp, bd):
    return pl.pallas_call(
        _agg_decode_body,
        grid=(_N // _BN,),
        in_specs=[
            pl.BlockSpec((1, _BN, _P), lambda i: (0, i, 0)),
            pl.BlockSpec((1, _BN, _P), lambda i: (1, i, 0)),
            pl.BlockSpec((_P, 8), lambda i: (0, 0)),
            pl.BlockSpec((1, 8), lambda i: (0, 0)),
        ],
        out_specs=pl.BlockSpec((_BN, 8), lambda i: (i, 0)),
        out_shape=jax.ShapeDtypeStruct((_N, 8), jnp.float32),
    )(parts, parts, clsp, bd)


def _sc_segment_sum(h_pad, edge_index, zeros_blk):
    mesh = plsc.VectorSubcoreMesh(core_axis_name="c", subcore_axis_name="s")

    @functools.partial(
        pl.kernel,
        mesh=mesh,
        compiler_params=pltpu.CompilerParams(use_tc_tiling_on_sc=False),
        out_type=jax.ShapeDtypeStruct((_NC, _NA, _P), jnp.float32),
        scratch_types=[
            pltpu.VMEM((1, _EB), jnp.int32),
            pltpu.VMEM((1, _EB), jnp.int32),
            pltpu.VMEM((_EB, _P), jnp.float32),
            pltpu.VMEM_SHARED((_NA, _P), jnp.float32),
            pltpu.SemaphoreType.DMA,
        ],
    )
    def k(h_hbm, ei_hbm, z_hbm, out_hbm, src_v, dst_v, rows_v, acc, sem):
        c = lax.axis_index("c")
        s = lax.axis_index("s")
        wid = s * _NC + c
        pltpu.sync_copy(z_hbm, acc.at[pl.ds(s * _RPS, _RPS)])
        plsc.subcore_barrier()

        def body(j, carry):
            step = wid + _NW * j

            @pl.when(step < _STEPS)
            def _():
                off = step * _EB
                pltpu.sync_copy(ei_hbm.at[0, pl.ds(off, _EB)], src_v.at[0])
                pltpu.sync_copy(ei_hbm.at[1, pl.ds(off, _EB)], dst_v.at[0])
                pltpu.async_copy(h_hbm.at[src_v.at[0]], rows_v, sem).wait()
                pltpu.sync_copy(rows_v, acc.at[dst_v.at[0]], add=True)

            return carry

        lax.fori_loop(0, _SPW, body, 0)
        plsc.subcore_barrier()
        pltpu.sync_copy(acc.at[pl.ds(s * _RPS, _RPS)],
                        out_hbm.at[c, pl.ds(s * _RPS, _RPS)])

    return k(h_pad, edge_index, zeros_blk)


def kernel(node_feat, edge_index, W1, b1, s1, W2, b2, s2, cls, bias_dec):
    f32 = jnp.float32
    w1s = jnp.zeros((128, _P), f32).at[:, :_DH].set(W1[:, 1:].T)
    w1t = jnp.zeros((1, _P), f32).at[0, :_DH].set(W1[:, 0])
    b1p = jnp.zeros((1, _P), f32).at[0, :_DH].set(b1)
    w2p = jnp.zeros((_P, _P), f32).at[:_DH, :_DH].set(W2.T)
    b2p = jnp.zeros((1, _P), f32).at[0, :_DH].set(b2)
    clsp = jnp.zeros((_P, 8), f32).at[:_DH, :7].set(cls.T)
    bdp = jnp.zeros((1, 8), f32).at[0, :7].set(bias_dec)
    s1a = jnp.reshape(s1, (1, 1)).astype(f32)
    s2a = jnp.reshape(s2, (1, 1)).astype(f32)
    zeros_blk = jnp.zeros((_RPS, _P), f32)

    h1 = _tc_expmap_linear(node_feat, w1s, w1t, b1p, s1a)
    a1 = _sc_segment_sum(h1, edge_index, zeros_blk)
    h2 = _tc_agg_linear(a1, w2p, b2p, s2a)
    a2 = _sc_segment_sum(h2, edge_index, zeros_blk)
    out8 = _tc_agg_decode(a2, clsp, bdp)
    return out8[:, :7]


# speedup vs baseline: 7.4189x; 1.3297x over previous
import functools

import jax
import jax.numpy as jnp
from jax import lax
from jax.experimental import pallas as pl
from jax.experimental.pallas import tpu as pltpu
from jax.experimental.pallas import tpu_sc as plsc

_N = 10000
_E = 320000
_DH = 129
_P = 144
_EB = 40
_NC = 2
_NS = 16
_NW = _NC * _NS
_NB = _E // (_NW * _EB)
_NA = 10240
_RPS = _NA // _NS
_BN = 1000


def _lorentz_normalize(y, s_raw):
    sfac = jnp.minimum(jnp.exp(s_raw), 10.0)
    t = sfac / (1.0 + jnp.exp(-y[:, 0:1])) + 1.5
    sq = jnp.sum(y * y, axis=1, keepdims=True) - y[:, 0:1] * y[:, 0:1]
    sq = jnp.maximum(sq, 1e-8)
    fac = jnp.sqrt(jnp.maximum((t * t - 1.0) / sq, 1e-8))
    col = lax.broadcasted_iota(jnp.int32, y.shape, 1)
    return jnp.where(col == 0, t, y * fac)


def _expmap_linear_body(x_ref, w_ref, wt_ref, b_ref, s_ref, o_ref):
    x = x_ref[...]
    nrm = jnp.maximum(jnp.sqrt(jnp.sum(x * x, axis=1, keepdims=True)), 1e-8)
    e = jnp.exp(nrm)
    ei = 1.0 / e
    time = 0.5 * (e + ei)
    coef = (0.5 * (e - ei)) / nrm
    y = jnp.dot(coef * x, w_ref[...], preferred_element_type=jnp.float32)
    y = y + time * wt_ref[...] + b_ref[...]
    o_ref[...] = _lorentz_normalize(y, s_ref[0, 0])


def _agg_linear_body(p0_ref, p1_ref, w_ref, b_ref, s_ref, o_ref):
    sup = p0_ref[0] + p1_ref[0]
    t0 = sup[:, 0:1]
    inner = jnp.sum(sup * sup, axis=1, keepdims=True) - 2.0 * t0 * t0
    denom = jnp.sqrt(jnp.maximum(jnp.abs(inner), 1e-8))
    xr = jnp.maximum(sup / denom, 0.0)
    y = jnp.dot(xr, w_ref[...], preferred_element_type=jnp.float32) + b_ref[...]
    o_ref[...] = _lorentz_normalize(y, s_ref[0, 0])


def _agg_decode_body(p0_ref, p1_ref, cls_ref, b_ref, o_ref):
    sup = p0_ref[0] + p1_ref[0]
    t0 = sup[:, 0:1]
    inner = jnp.sum(sup * sup, axis=1, keepdims=True) - 2.0 * t0 * t0
    denom = jnp.sqrt(jnp.maximum(jnp.abs(inner), 1e-8))
    h = sup / denom
    col = lax.broadcasted_iota(jnp.int32, h.shape, 1)
    xm = jnp.where(col == 0, -h, h)
    o_ref[...] = (2.0 + 2.0 * jnp.dot(xm, cls_ref[...],
                                      preferred_element_type=jnp.float32)
                  + b_ref[...])


def _tc_expmap_linear(x, w, wt, b, s):
    return pl.pallas_call(
        _expmap_linear_body,
        grid=(_N // _BN,),
        in_specs=[
            pl.BlockSpec((_BN, 128), lambda i: (i, 0)),
            pl.BlockSpec((128, _P), lambda i: (0, 0)),
            pl.BlockSpec((1, _P), lambda i: (0, 0)),
            pl.BlockSpec((1, _P), lambda i: (0, 0)),
            pl.BlockSpec((1, 1), lambda i: (0, 0)),
        ],
        out_specs=pl.BlockSpec((_BN, _P), lambda i: (i, 0)),
        out_shape=jax.ShapeDtypeStruct((_N, _P), jnp.float32),
    )(x, w, wt, b, s)


def _tc_agg_linear(parts, w, b, s):
    return pl.pallas_call(
        _agg_linear_body,
        grid=(_N // _BN,),
        in_specs=[
            pl.BlockSpec((1, _BN, _P), lambda i: (0, i, 0)),
            pl.BlockSpec((1, _BN, _P), lambda i: (1, i, 0)),
            pl.BlockSpec((_P, _P), lambda i: (0, 0)),
            pl.BlockSpec((1, _P), lambda i: (0, 0)),
            pl.BlockSpec((1, 1), lambda i: (0, 0)),
        ],
        out_specs=pl.BlockSpec((_BN, _P), lambda i: (i, 0)),
        out_shape=jax.ShapeDtypeStruct((_N, _P), jnp.float32),
    )(parts, parts, w, b, s)


def _tc_agg_decode(parts, clsp, bd):
    return pl.pallas_call(
        _agg_decode_body,
        grid=(_N // _BN,),
        in_specs=[
            pl.BlockSpec((1, _BN, _P), lambda i: (0, i, 0)),
            pl.BlockSpec((1, _BN, _P), lambda i: (1, i, 0)),
            pl.BlockSpec((_P, 8), lambda i: (0, 0)),
            pl.BlockSpec((1, 8), lambda i: (0, 0)),
        ],
        out_specs=pl.BlockSpec((_BN, 8), lambda i: (i, 0)),
        out_shape=jax.ShapeDtypeStruct((_N, 8), jnp.float32),
    )(parts, parts, clsp, bd)


def _sc_segment_sum(h_pad, edge_index3, zeros_blk):
    mesh = plsc.VectorSubcoreMesh(core_axis_name="c", subcore_axis_name="s")

    @functools.partial(
        pl.kernel,
        mesh=mesh,
        compiler_params=pltpu.CompilerParams(use_tc_tiling_on_sc=False),
        out_type=jax.ShapeDtypeStruct((_NC, _NA, _P), jnp.float32),
        scratch_types=[
            pltpu.VMEM((_NB, _EB), jnp.int32),
            pltpu.VMEM((_NB, _EB), jnp.int32),
            pltpu.VMEM((_EB, _P), jnp.float32),
            pltpu.VMEM((_EB, _P), jnp.float32),
            pltpu.VMEM_SHARED((_NA, _P), jnp.float32),
            pltpu.SemaphoreType.DMA,
            pltpu.SemaphoreType.DMA,
        ],
    )
    def k(h_hbm, src_hbm, dst_hbm, z_hbm, out_hbm, src_v, dst_v, rows0, rows1,
          acc, sem0, sem1):
        c = lax.axis_index("c")
        s = lax.axis_index("s")
        wid = s * _NC + c
        pltpu.sync_copy(src_hbm.at[pl.ds(wid * _NB, _NB)], src_v)
        pltpu.sync_copy(dst_hbm.at[pl.ds(wid * _NB, _NB)], dst_v)
        pltpu.sync_copy(z_hbm, acc.at[pl.ds(s * _RPS, _RPS)])
        plsc.subcore_barrier()

        def gather(j, rows, sem):
            return pltpu.async_copy(h_hbm.at[src_v.at[j]], rows, sem)

        gather(0, rows0, sem0)

        def body(i, carry):
            s0 = 2 * i
            s1 = s0 + 1
            gather(s1, rows1, sem1)
            pltpu.make_async_copy(h_hbm.at[src_v.at[s0]], rows0, sem0).wait()
            pltpu.sync_copy(rows0, acc.at[dst_v.at[s0]], add=True)

            @pl.when(s0 + 2 < _NB)
            def _():
                gather(s0 + 2, rows0, sem0)

            pltpu.make_async_copy(h_hbm.at[src_v.at[s1]], rows1, sem1).wait()
            pltpu.sync_copy(rows1, acc.at[dst_v.at[s1]], add=True)
            return carry

        lax.fori_loop(0, _NB // 2, body, 0)
        plsc.subcore_barrier()
        pltpu.sync_copy(acc.at[pl.ds(s * _RPS, _RPS)],
                        out_hbm.at[c, pl.ds(s * _RPS, _RPS)])

    return k(h_pad, edge_index3[0], edge_index3[1], zeros_blk)


def kernel(node_feat, edge_index, W1, b1, s1, W2, b2, s2, cls, bias_dec):
    f32 = jnp.float32
    w1s = jnp.zeros((128, _P), f32).at[:, :_DH].set(W1[:, 1:].T)
    w1t = jnp.zeros((1, _P), f32).at[0, :_DH].set(W1[:, 0])
    b1p = jnp.zeros((1, _P), f32).at[0, :_DH].set(b1)
    w2p = jnp.zeros((_P, _P), f32).at[:_DH, :_DH].set(W2.T)
    b2p = jnp.zeros((1, _P), f32).at[0, :_DH].set(b2)
    clsp = jnp.zeros((_P, 8), f32).at[:_DH, :7].set(cls.T)
    bdp = jnp.zeros((1, 8), f32).at[0, :7].set(bias_dec)
    s1a = jnp.reshape(s1, (1, 1)).astype(f32)
    s2a = jnp.reshape(s2, (1, 1)).astype(f32)
    zeros_blk = jnp.zeros((_RPS, _P), f32)
    ei3 = jnp.reshape(edge_index, (2, _NW * _NB, _EB))

    h1 = _tc_expmap_linear(node_feat, w1s, w1t, b1p, s1a)
    a1 = _sc_segment_sum(h1, ei3, zeros_blk)
    h2 = _tc_agg_linear(a1, w2p, b2p, s2a)
    a2 = _sc_segment_sum(h2, ei3, zeros_blk)
    out8 = _tc_agg_decode(a2, clsp, bdp)
    return out8[:, :7]


# speedup vs baseline: 8.8950x; 1.1990x over previous
import functools

import jax
import jax.numpy as jnp
from jax import lax
from jax.experimental import pallas as pl
from jax.experimental.pallas import tpu as pltpu
from jax.experimental.pallas import tpu_sc as plsc

_N = 10000
_E = 320000
_DH = 129
_P = 144
_EB = 80
_NC = 2
_NS = 16
_NW = _NC * _NS
_NB = _E // (_NW * _EB)
_NA = 10240
_RPS = _NA // _NS
_BN = 1000


def _lorentz_normalize(y, s_raw):
    sfac = jnp.minimum(jnp.exp(s_raw), 10.0)
    t = sfac / (1.0 + jnp.exp(-y[:, 0:1])) + 1.5
    sq = jnp.sum(y * y, axis=1, keepdims=True) - y[:, 0:1] * y[:, 0:1]
    sq = jnp.maximum(sq, 1e-8)
    fac = jnp.sqrt(jnp.maximum((t * t - 1.0) / sq, 1e-8))
    col = lax.broadcasted_iota(jnp.int32, y.shape, 1)
    return jnp.where(col == 0, t, y * fac)


def _expmap_linear_body(x_ref, w_ref, wt_ref, b_ref, s_ref, o_ref):
    x = x_ref[...]
    nrm = jnp.maximum(jnp.sqrt(jnp.sum(x * x, axis=1, keepdims=True)), 1e-8)
    e = jnp.exp(nrm)
    ei = 1.0 / e
    time = 0.5 * (e + ei)
    coef = (0.5 * (e - ei)) / nrm
    y = jnp.dot(coef * x, w_ref[...], preferred_element_type=jnp.float32)
    y = y + time * wt_ref[...] + b_ref[...]
    o_ref[...] = _lorentz_normalize(y, s_ref[0, 0])


def _agg_linear_body(p0_ref, p1_ref, w_ref, b_ref, s_ref, o_ref):
    sup = p0_ref[0] + p1_ref[0]
    t0 = sup[:, 0:1]
    inner = jnp.sum(sup * sup, axis=1, keepdims=True) - 2.0 * t0 * t0
    denom = jnp.sqrt(jnp.maximum(jnp.abs(inner), 1e-8))
    xr = jnp.maximum(sup / denom, 0.0)
    y = jnp.dot(xr, w_ref[...], preferred_element_type=jnp.float32) + b_ref[...]
    o_ref[...] = _lorentz_normalize(y, s_ref[0, 0])


def _agg_decode_body(p0_ref, p1_ref, cls_ref, b_ref, o_ref):
    sup = p0_ref[0] + p1_ref[0]
    t0 = sup[:, 0:1]
    inner = jnp.sum(sup * sup, axis=1, keepdims=True) - 2.0 * t0 * t0
    denom = jnp.sqrt(jnp.maximum(jnp.abs(inner), 1e-8))
    h = sup / denom
    col = lax.broadcasted_iota(jnp.int32, h.shape, 1)
    xm = jnp.where(col == 0, -h, h)
    o_ref[...] = (2.0 + 2.0 * jnp.dot(xm, cls_ref[...],
                                      preferred_element_type=jnp.float32)
                  + b_ref[...])


def _tc_expmap_linear(x, w, wt, b, s):
    return pl.pallas_call(
        _expmap_linear_body,
        grid=(_N // _BN,),
        in_specs=[
            pl.BlockSpec((_BN, 128), lambda i: (i, 0)),
            pl.BlockSpec((128, _P), lambda i: (0, 0)),
            pl.BlockSpec((1, _P), lambda i: (0, 0)),
            pl.BlockSpec((1, _P), lambda i: (0, 0)),
            pl.BlockSpec((1, 1), lambda i: (0, 0)),
        ],
        out_specs=pl.BlockSpec((_BN, _P), lambda i: (i, 0)),
        out_shape=jax.ShapeDtypeStruct((_N, _P), jnp.float32),
    )(x, w, wt, b, s)


def _tc_agg_linear(parts, w, b, s):
    return pl.pallas_call(
        _agg_linear_body,
        grid=(_N // _BN,),
        in_specs=[
            pl.BlockSpec((1, _BN, _P), lambda i: (0, i, 0)),
            pl.BlockSpec((1, _BN, _P), lambda i: (1, i, 0)),
            pl.BlockSpec((_P, _P), lambda i: (0, 0)),
            pl.BlockSpec((1, _P), lambda i: (0, 0)),
            pl.BlockSpec((1, 1), lambda i: (0, 0)),
        ],
        out_specs=pl.BlockSpec((_BN, _P), lambda i: (i, 0)),
        out_shape=jax.ShapeDtypeStruct((_N, _P), jnp.float32),
    )(parts, parts, w, b, s)


def _tc_agg_decode(parts, clsp, bd):
    return pl.pallas_call(
        _agg_decode_body,
        grid=(_N // _BN,),
        in_specs=[
            pl.BlockSpec((1, _BN, _P), lambda i: (0, i, 0)),
            pl.BlockSpec((1, _BN, _P), lambda i: (1, i, 0)),
            pl.BlockSpec((_P, 8), lambda i: (0, 0)),
            pl.BlockSpec((1, 8), lambda i: (0, 0)),
        ],
        out_specs=pl.BlockSpec((_BN, 8), lambda i: (i, 0)),
        out_shape=jax.ShapeDtypeStruct((_N, 8), jnp.float32),
    )(parts, parts, clsp, bd)


def _sc_segment_sum(h_pad, edge_index3, zeros_blk):
    mesh = plsc.VectorSubcoreMesh(core_axis_name="c", subcore_axis_name="s")

    @functools.partial(
        pl.kernel,
        mesh=mesh,
        compiler_params=pltpu.CompilerParams(use_tc_tiling_on_sc=False),
        out_type=jax.ShapeDtypeStruct((_NC, _NA, _P), jnp.float32),
        scratch_types=[
            pltpu.VMEM((2, _EB), jnp.int32),
            pltpu.VMEM((_NB, _EB), jnp.int32),
            pltpu.VMEM((_EB, _P), jnp.float32),
            pltpu.VMEM((_EB, _P), jnp.float32),
            pltpu.VMEM_SHARED((_NA, _P), jnp.float32),
            pltpu.SemaphoreType.DMA,
            pltpu.SemaphoreType.DMA,
            pltpu.SemaphoreType.DMA,
            pltpu.SemaphoreType.DMA,
        ],
    )
    def k(h_hbm, src_hbm, dst_hbm, z_hbm, out_hbm, src_v, dst_v, rows0, rows1,
          acc, gsem0, gsem1, isem0, isem1):
        c = lax.axis_index("c")
        s = lax.axis_index("s")
        wid = s * _NC + c
        pltpu.sync_copy(dst_hbm.at[pl.ds(wid * _NB, _NB)], dst_v)
        pltpu.sync_copy(z_hbm, acc.at[pl.ds(s * _RPS, _RPS)])
        plsc.subcore_barrier()

        def idx_cp(j, b, isem):
            return pltpu.make_async_copy(src_hbm.at[pl.ds(wid * _NB + j, 1)],
                                         src_v.at[pl.ds(b, 1)], isem)

        def g_cp(b, rows, gsem):
            return pltpu.make_async_copy(h_hbm.at[src_v.at[b]], rows, gsem)

        def scatter(j, rows):
            pltpu.sync_copy(rows, acc.at[dst_v.at[j]], add=True)

        idx_cp(0, 0, isem0).start()
        idx_cp(0, 0, isem0).wait()
        g_cp(0, rows0, gsem0).start()
        idx_cp(1, 1, isem1).start()

        def body(i, carry):
            s0 = 2 * i
            s1 = s0 + 1
            idx_cp(s1, 1, isem1).wait()
            g_cp(1, rows1, gsem1).start()
            g_cp(0, rows0, gsem0).wait()
            idx_cp(s0 + 2, 0, isem0).start()
            scatter(s0, rows0)
            idx_cp(s0 + 2, 0, isem0).wait()
            g_cp(0, rows0, gsem0).start()
            g_cp(1, rows1, gsem1).wait()

            @pl.when(s1 + 2 < _NB)
            def _():
                idx_cp(s1 + 2, 1, isem1).start()

            scatter(s1, rows1)
            return carry

        lax.fori_loop(0, (_NB - 1) // 2, body, 0)
        g_cp(0, rows0, gsem0).wait()
        scatter(_NB - 1, rows0)
        plsc.subcore_barrier()
        pltpu.sync_copy(acc.at[pl.ds(s * _RPS, _RPS)],
                        out_hbm.at[c, pl.ds(s * _RPS, _RPS)])

    return k(h_pad, edge_index3[0], edge_index3[1], zeros_blk)


def kernel(node_feat, edge_index, W1, b1, s1, W2, b2, s2, cls, bias_dec):
    f32 = jnp.float32
    w1s = jnp.zeros((128, _P), f32).at[:, :_DH].set(W1[:, 1:].T)
    w1t = jnp.zeros((1, _P), f32).at[0, :_DH].set(W1[:, 0])
    b1p = jnp.zeros((1, _P), f32).at[0, :_DH].set(b1)
    w2p = jnp.zeros((_P, _P), f32).at[:_DH, :_DH].set(W2.T)
    b2p = jnp.zeros((1, _P), f32).at[0, :_DH].set(b2)
    clsp = jnp.zeros((_P, 8), f32).at[:_DH, :7].set(cls.T)
    bdp = jnp.zeros((1, 8), f32).at[0, :7].set(bias_dec)
    s1a = jnp.reshape(s1, (1, 1)).astype(f32)
    s2a = jnp.reshape(s2, (1, 1)).astype(f32)
    zeros_blk = jnp.zeros((_RPS, _P), f32)
    ei3 = jnp.reshape(edge_index, (2, _NW * _NB, _EB))

    h1 = _tc_expmap_linear(node_feat, w1s, w1t, b1p, s1a)
    a1 = _sc_segment_sum(h1, ei3, zeros_blk)
    h2 = _tc_agg_linear(a1, w2p, b2p, s2a)
    a2 = _sc_segment_sum(h2, ei3, zeros_blk)
    out8 = _tc_agg_decode(a2, clsp, bdp)
    return out8[:, :7]


# speedup vs baseline: 10.2540x; 1.1528x over previous
import functools

import jax
import jax.numpy as jnp
from jax import lax
from jax.experimental import pallas as pl
from jax.experimental.pallas import tpu as pltpu
from jax.experimental.pallas import tpu_sc as plsc

_N = 10000
_E = 320000
_PS = 128
_PT = 8
_EB = 80
_NC = 2
_NS = 16
_NW = _NC * _NS
_NB = _E // (_NW * _EB)
_NA = 10240
_RPS = _NA // _NS
_BN = 1000


def _lorentz_tail(y, s_raw):
    sfac = jnp.minimum(jnp.exp(s_raw), 10.0)
    t = sfac / (1.0 + jnp.exp(-y[:, 0:1])) + 1.5
    sq = jnp.sum(y * y, axis=1, keepdims=True) - y[:, 0:1] * y[:, 0:1]
    sq = jnp.maximum(sq, 1e-8)
    fac = jnp.sqrt(jnp.maximum((t * t - 1.0) / sq, 1e-8))
    space = y[:, 1:129] * fac
    col = lax.broadcasted_iota(jnp.int32, (y.shape[0], _PT), 1)
    time8 = jnp.where(col == 0, t, 0.0)
    return space, time8


def _nt_dot(x, w):
    return lax.dot_general(x, w, (((1,), (1,)), ((), ())),
                           preferred_element_type=jnp.float32)


def _expmap_linear_body(x_ref, w_ref, b_ref, s_ref, osp_ref, ot_ref):
    x = x_ref[...]
    nrm = jnp.maximum(jnp.sqrt(jnp.sum(x * x, axis=1, keepdims=True)), 1e-8)
    e = jnp.exp(nrm)
    ei = 1.0 / e
    time = 0.5 * (e + ei)
    coef = (0.5 * (e - ei)) / nrm
    w = w_ref[...]
    y = _nt_dot(coef * x, w[:, 1:]) + time * w[:, 0] + b_ref[...]
    osp_ref[...], ot_ref[...] = _lorentz_tail(y, s_ref[0, 0])


def _agg_linear_body(p0_ref, p1_ref, q0_ref, q1_ref, w_ref, b_ref, s_ref,
                     osp_ref, ot_ref):
    sup = p0_ref[0] + p1_ref[0]
    t0 = (q0_ref[0] + q1_ref[0])[:, 0:1]
    inner = jnp.sum(sup * sup, axis=1, keepdims=True) - t0 * t0
    denom = jnp.sqrt(jnp.maximum(jnp.abs(inner), 1e-8))
    xr = jnp.maximum(sup / denom, 0.0)
    xt = jnp.maximum(t0 / denom, 0.0)
    w = w_ref[...]
    y = _nt_dot(xr, w[:, 1:]) + xt * w[:, 0] + b_ref[...]
    osp_ref[...], ot_ref[...] = _lorentz_tail(y, s_ref[0, 0])


def _agg_decode_body(p0_ref, p1_ref, q0_ref, q1_ref, cls_ref, b_ref, o_ref):
    sup = p0_ref[0] + p1_ref[0]
    t0 = (q0_ref[0] + q1_ref[0])[:, 0:1]
    inner = jnp.sum(sup * sup, axis=1, keepdims=True) - t0 * t0
    denom = jnp.sqrt(jnp.maximum(jnp.abs(inner), 1e-8))
    h = sup / denom
    ht = t0 / denom
    cw = cls_ref[...]
    y = _nt_dot(h, cw[:, 1:]) - ht * cw[:, 0]
    o_ref[...] = 2.0 + 2.0 * y + b_ref[...]


def _tc_expmap_linear(x, w, b, s):
    return pl.pallas_call(
        _expmap_linear_body,
        grid=(_N // _BN,),
        in_specs=[
            pl.BlockSpec((_BN, _PS), lambda i: (i, 0)),
            pl.BlockSpec((129, 129), lambda i: (0, 0)),
            pl.BlockSpec((1, 129), lambda i: (0, 0)),
            pl.BlockSpec((1, 1), lambda i: (0, 0)),
        ],
        out_specs=[pl.BlockSpec((_BN, _PS), lambda i: (i, 0)),
                   pl.BlockSpec((_BN, _PT), lambda i: (i, 0))],
        out_shape=[jax.ShapeDtypeStruct((_N, _PS), jnp.float32),
                   jax.ShapeDtypeStruct((_N, _PT), jnp.float32)],
    )(x, w, b, s)


def _tc_agg_linear(psp, pt, w, b, s):
    return pl.pallas_call(
        _agg_linear_body,
        grid=(_N // _BN,),
        in_specs=[
            pl.BlockSpec((1, _BN, _PS), lambda i: (0, i, 0)),
            pl.BlockSpec((1, _BN, _PS), lambda i: (1, i, 0)),
            pl.BlockSpec((1, _BN, _PT), lambda i: (0, i, 0)),
            pl.BlockSpec((1, _BN, _PT), lambda i: (1, i, 0)),
            pl.BlockSpec((129, 129), lambda i: (0, 0)),
            pl.BlockSpec((1, 129), lambda i: (0, 0)),
            pl.BlockSpec((1, 1), lambda i: (0, 0)),
        ],
        out_specs=[pl.BlockSpec((_BN, _PS), lambda i: (i, 0)),
                   pl.BlockSpec((_BN, _PT), lambda i: (i, 0))],
        out_shape=[jax.ShapeDtypeStruct((_N, _PS), jnp.float32),
                   jax.ShapeDtypeStruct((_N, _PT), jnp.float32)],
    )(psp, psp, pt, pt, w, b, s)


def _tc_agg_decode(psp, pt, clsw, bd):
    return pl.pallas_call(
        _agg_decode_body,
        grid=(_N // _BN,),
        in_specs=[
            pl.BlockSpec((1, _BN, _PS), lambda i: (0, i, 0)),
            pl.BlockSpec((1, _BN, _PS), lambda i: (1, i, 0)),
            pl.BlockSpec((1, _BN, _PT), lambda i: (0, i, 0)),
            pl.BlockSpec((1, _BN, _PT), lambda i: (1, i, 0)),
            pl.BlockSpec((7, 129), lambda i: (0, 0)),
            pl.BlockSpec((1, 7), lambda i: (0, 0)),
        ],
        out_specs=pl.BlockSpec((_BN, 7), lambda i: (i, 0)),
        out_shape=jax.ShapeDtypeStruct((_N, 7), jnp.float32),
    )(psp, psp, pt, pt, clsw, bd)


def _sc_segment_sum(h_sp, h_t, ei3, z_sp, z_t):
    mesh = plsc.VectorSubcoreMesh(core_axis_name="c", subcore_axis_name="s")

    @functools.partial(
        pl.kernel,
        mesh=mesh,
        compiler_params=pltpu.CompilerParams(use_tc_tiling_on_sc=False),
        out_type=[jax.ShapeDtypeStruct((_NC, _NA, _PS), jnp.float32),
                  jax.ShapeDtypeStruct((_NC, _NA, _PT), jnp.float32)],
        scratch_types=[
            pltpu.VMEM((2, _EB), jnp.int32),
            pltpu.VMEM((_NB, _EB), jnp.int32),
            pltpu.VMEM((_EB, _PS), jnp.float32),
            pltpu.VMEM((_EB, _PS), jnp.float32),
            pltpu.VMEM((_EB, _PT), jnp.float32),
            pltpu.VMEM((_EB, _PT), jnp.float32),
            pltpu.VMEM_SHARED((_NA, _PS), jnp.float32),
            pltpu.VMEM_SHARED((_NA, _PT), jnp.float32),
            pltpu.SemaphoreType.DMA,
            pltpu.SemaphoreType.DMA,
            pltpu.SemaphoreType.DMA,
            pltpu.SemaphoreType.DMA,
        ],
    )
    def k(hsp_hbm, ht_hbm, ei_hbm, zsp_hbm, zt_hbm, osp_hbm, ot_hbm,
          src_v, dst_v, rows0, rows1, trow0, trow1, accs, acct,
          gsem0, gsem1, isem0, isem1):
        c = lax.axis_index("c")
        s = lax.axis_index("s")
        wid = s * _NC + c
        pltpu.sync_copy(ei_hbm.at[1, pl.ds(wid * _NB, _NB)], dst_v)
        pltpu.sync_copy(zsp_hbm, accs.at[pl.ds(s * _RPS, _RPS)])
        pltpu.sync_copy(zt_hbm, acct.at[pl.ds(s * _RPS, _RPS)])
        plsc.subcore_barrier()

        def idx_cp(j, b, isem):
            return pltpu.make_async_copy(
                ei_hbm.at[0, pl.ds(wid * _NB + j, 1)],
                src_v.at[pl.ds(b, 1)], isem)

        def g_sp(b, rows, gsem):
            return pltpu.make_async_copy(hsp_hbm.at[src_v.at[b]], rows, gsem)

        def g_t(b, trow, gsem):
            return pltpu.make_async_copy(ht_hbm.at[src_v.at[b]], trow, gsem)

        def g_start(b, rows, trow, gsem):
            g_sp(b, rows, gsem).start()
            g_t(b, trow, gsem).start()

        def g_wait(b, rows, trow, gsem):
            g_sp(b, rows, gsem).wait()
            g_t(b, trow, gsem).wait()

        def scatter(j, rows, trow):
            pltpu.sync_copy(rows, accs.at[dst_v.at[j]], add=True)
            pltpu.sync_copy(trow, acct.at[dst_v.at[j]], add=True)

        idx_cp(0, 0, isem0).start()
        idx_cp(0, 0, isem0).wait()
        g_start(0, rows0, trow0, gsem0)
        idx_cp(1, 1, isem1).start()

        def body(i, carry):
            s0 = 2 * i
            s1 = s0 + 1
            idx_cp(s1, 1, isem1).wait()
            g_start(1, rows1, trow1, gsem1)
            g_wait(0, rows0, trow0, gsem0)
            idx_cp(s0 + 2, 0, isem0).start()
            scatter(s0, rows0, trow0)
            idx_cp(s0 + 2, 0, isem0).wait()
            g_start(0, rows0, trow0, gsem0)
            g_wait(1, rows1, trow1, gsem1)

            @pl.when(s1 + 2 < _NB)
            def _():
                idx_cp(s1 + 2, 1, isem1).start()

            scatter(s1, rows1, trow1)
            return carry

        lax.fori_loop(0, (_NB - 1) // 2, body, 0)
        g_wait(0, rows0, trow0, gsem0)
        scatter(_NB - 1, rows0, trow0)
        plsc.subcore_barrier()
        pltpu.sync_copy(accs.at[pl.ds(s * _RPS, _RPS)],
                        osp_hbm.at[c, pl.ds(s * _RPS, _RPS)])
        pltpu.sync_copy(acct.at[pl.ds(s * _RPS, _RPS)],
                        ot_hbm.at[c, pl.ds(s * _RPS, _RPS)])

    return k(h_sp, h_t, ei3, z_sp, z_t)


def kernel(node_feat, edge_index, W1, b1, s1, W2, b2, s2, cls, bias_dec):
    f32 = jnp.float32
    b1r = jnp.reshape(b1, (1, 129))
    b2r = jnp.reshape(b2, (1, 129))
    bdr = jnp.reshape(bias_dec, (1, 7))
    s1a = jnp.reshape(s1, (1, 1)).astype(f32)
    s2a = jnp.reshape(s2, (1, 1)).astype(f32)
    ei3 = jnp.reshape(edge_index, (2, _NW * _NB, _EB))
    z_sp = jnp.zeros((_RPS, _PS), f32)
    z_t = jnp.zeros((_RPS, _PT), f32)

    h1s, h1t = _tc_expmap_linear(node_feat, W1, b1r, s1a)
    a1s, a1t = _sc_segment_sum(h1s, h1t, ei3, z_sp, z_t)
    h2s, h2t = _tc_agg_linear(a1s, a1t, W2, b2r, s2a)
    a2s, a2t = _sc_segment_sum(h2s, h2t, ei3, z_sp, z_t)
    return _tc_agg_decode(a2s, a2t, cls, bdr)


# speedup vs baseline: 10.6308x; 1.0367x over previous
import functools

import jax
import jax.numpy as jnp
from jax import lax
from jax.experimental import pallas as pl
from jax.experimental.pallas import tpu as pltpu
from jax.experimental.pallas import tpu_sc as plsc

_N = 10000
_E = 320000
_PS = 128
_PT = 8
_EB = 80
_NC = 2
_NS = 16
_NW = _NC * _NS
_NB = _E // (_NW * _EB)
_NA = 10112
_RPS = _NA // _NS
_BN = 1000


def _lorentz_tail(y, s_raw):
    sfac = jnp.minimum(jnp.exp(s_raw), 10.0)
    t = sfac / (1.0 + jnp.exp(-y[:, 0:1])) + 1.5
    sq = jnp.sum(y * y, axis=1, keepdims=True) - y[:, 0:1] * y[:, 0:1]
    sq = jnp.maximum(sq, 1e-8)
    fac = jnp.sqrt(jnp.maximum((t * t - 1.0) / sq, 1e-8))
    space = y[:, 1:129] * fac
    col = lax.broadcasted_iota(jnp.int32, (y.shape[0], _PT), 1)
    time8 = jnp.where(col == 0, t, 0.0)
    return space, time8


def _nt_dot(x, w):
    return lax.dot_general(x, w, (((1,), (1,)), ((), ())),
                           preferred_element_type=jnp.float32)


def _expmap_linear_body(x_ref, w_ref, b_ref, s_ref, osp_ref, ot_ref):
    x = x_ref[...]
    nrm = jnp.maximum(jnp.sqrt(jnp.sum(x * x, axis=1, keepdims=True)), 1e-8)
    e = jnp.exp(nrm)
    ei = 1.0 / e
    time = 0.5 * (e + ei)
    coef = (0.5 * (e - ei)) / nrm
    w = w_ref[...]
    y = _nt_dot(coef * x, w[:, 1:]) + time * w[:, 0] + b_ref[...]
    osp_ref[...], ot_ref[...] = _lorentz_tail(y, s_ref[0, 0])


def _agg_linear_body(p0_ref, p1_ref, q0_ref, q1_ref, w_ref, b_ref, s_ref,
                     osp_ref, ot_ref):
    sup = p0_ref[0] + p1_ref[0]
    t0 = (q0_ref[0] + q1_ref[0])[:, 0:1]
    inner = jnp.sum(sup * sup, axis=1, keepdims=True) - t0 * t0
    denom = jnp.sqrt(jnp.maximum(jnp.abs(inner), 1e-8))
    xr = jnp.maximum(sup / denom, 0.0)
    xt = jnp.maximum(t0 / denom, 0.0)
    w = w_ref[...]
    y = _nt_dot(xr, w[:, 1:]) + xt * w[:, 0] + b_ref[...]
    osp_ref[...], ot_ref[...] = _lorentz_tail(y, s_ref[0, 0])


def _agg_decode_body(p0_ref, p1_ref, q0_ref, q1_ref, cls_ref, b_ref, o_ref):
    sup = p0_ref[0] + p1_ref[0]
    t0 = (q0_ref[0] + q1_ref[0])[:, 0:1]
    inner = jnp.sum(sup * sup, axis=1, keepdims=True) - t0 * t0
    denom = jnp.sqrt(jnp.maximum(jnp.abs(inner), 1e-8))
    h = sup / denom
    ht = t0 / denom
    cw = cls_ref[...]
    y = _nt_dot(h, cw[:, 1:]) - ht * cw[:, 0]
    o_ref[...] = 2.0 + 2.0 * y + b_ref[...]


def _tc_expmap_linear(x, w, b, s):
    return pl.pallas_call(
        _expmap_linear_body,
        grid=(_N // _BN,),
        in_specs=[
            pl.BlockSpec((_BN, _PS), lambda i: (i, 0)),
            pl.BlockSpec((129, 129), lambda i: (0, 0)),
            pl.BlockSpec((1, 129), lambda i: (0, 0)),
            pl.BlockSpec((1, 1), lambda i: (0, 0)),
        ],
        out_specs=[pl.BlockSpec((_BN, _PS), lambda i: (i, 0)),
                   pl.BlockSpec((_BN, _PT), lambda i: (i, 0))],
        out_shape=[jax.ShapeDtypeStruct((_N, _PS), jnp.float32),
                   jax.ShapeDtypeStruct((_N, _PT), jnp.float32)],
    )(x, w, b, s)


def _tc_agg_linear(psp, pt, w, b, s):
    return pl.pallas_call(
        _agg_linear_body,
        grid=(_N // _BN,),
        in_specs=[
            pl.BlockSpec((1, _BN, _PS), lambda i: (0, i, 0)),
            pl.BlockSpec((1, _BN, _PS), lambda i: (1, i, 0)),
            pl.BlockSpec((1, _BN, _PT), lambda i: (0, i, 0)),
            pl.BlockSpec((1, _BN, _PT), lambda i: (1, i, 0)),
            pl.BlockSpec((129, 129), lambda i: (0, 0)),
            pl.BlockSpec((1, 129), lambda i: (0, 0)),
            pl.BlockSpec((1, 1), lambda i: (0, 0)),
        ],
        out_specs=[pl.BlockSpec((_BN, _PS), lambda i: (i, 0)),
                   pl.BlockSpec((_BN, _PT), lambda i: (i, 0))],
        out_shape=[jax.ShapeDtypeStruct((_N, _PS), jnp.float32),
                   jax.ShapeDtypeStruct((_N, _PT), jnp.float32)],
    )(psp, psp, pt, pt, w, b, s)


def _tc_agg_decode(psp, pt, clsw, bd):
    return pl.pallas_call(
        _agg_decode_body,
        grid=(_N // _BN,),
        in_specs=[
            pl.BlockSpec((1, _BN, _PS), lambda i: (0, i, 0)),
            pl.BlockSpec((1, _BN, _PS), lambda i: (1, i, 0)),
            pl.BlockSpec((1, _BN, _PT), lambda i: (0, i, 0)),
            pl.BlockSpec((1, _BN, _PT), lambda i: (1, i, 0)),
            pl.BlockSpec((7, 129), lambda i: (0, 0)),
            pl.BlockSpec((1, 7), lambda i: (0, 0)),
        ],
        out_specs=pl.BlockSpec((_BN, 7), lambda i: (i, 0)),
        out_shape=jax.ShapeDtypeStruct((_N, 7), jnp.float32),
    )(psp, psp, pt, pt, clsw, bd)


def _sc_segment_sum(h_sp, h_t, ei3, z_sp, z_t):
    mesh = plsc.VectorSubcoreMesh(core_axis_name="c", subcore_axis_name="s")

    @functools.partial(
        pl.kernel,
        mesh=mesh,
        compiler_params=pltpu.CompilerParams(use_tc_tiling_on_sc=False),
        out_type=[jax.ShapeDtypeStruct((_NC, _NA, _PS), jnp.float32),
                  jax.ShapeDtypeStruct((_NC, _NA, _PT), jnp.float32)],
        scratch_types=[
            pltpu.VMEM((4, _EB), jnp.int32),
            pltpu.VMEM((4, _EB), jnp.int32),
            pltpu.VMEM((4, _EB, _PS), jnp.float32),
            pltpu.VMEM((4, _EB, _PT), jnp.float32),
            pltpu.VMEM_SHARED((_NA, _PS), jnp.float32),
            pltpu.VMEM_SHARED((_NA, _PT), jnp.float32),
            [pltpu.SemaphoreType.DMA] * 4,
            [pltpu.SemaphoreType.DMA] * 4,
            [pltpu.SemaphoreType.DMA] * 4,
            [pltpu.SemaphoreType.DMA] * 4,
        ],
    )
    def k(hsp_hbm, ht_hbm, ei_hbm, zsp_hbm, zt_hbm, osp_hbm, ot_hbm,
          src_v, dst_v, rows_v, trow_v, accs, acct,
          gsem, ssem, isem, dsem):
        c = lax.axis_index("c")
        s = lax.axis_index("s")
        wid = s * _NC + c
        pltpu.sync_copy(zsp_hbm, accs.at[pl.ds(s * _RPS, _RPS)])
        pltpu.sync_copy(zt_hbm, acct.at[pl.ds(s * _RPS, _RPS)])
        plsc.subcore_barrier()

        def sidx_cp(j, b):
            return pltpu.make_async_copy(
                ei_hbm.at[0, pl.ds(wid * _NB + j, 1)],
                src_v.at[pl.ds(b, 1)], isem[b])

        def didx_cp(j, b):
            return pltpu.make_async_copy(
                ei_hbm.at[1, pl.ds(wid * _NB + j, 1)],
                dst_v.at[pl.ds(b, 1)], dsem[b])

        def g_sp(b):
            return pltpu.make_async_copy(hsp_hbm.at[src_v.at[b]],
                                         rows_v.at[b], gsem[b])

        def g_t(b):
            return pltpu.make_async_copy(ht_hbm.at[src_v.at[b]],
                                         trow_v.at[b], gsem[b])

        def s_sp(b):
            return pltpu.make_async_copy(rows_v.at[b], accs.at[dst_v.at[b]],
                                         ssem[b])

        def s_t(b):
            return pltpu.make_async_copy(trow_v.at[b], acct.at[dst_v.at[b]],
                                         ssem[b])

        for b in range(4):
            sidx_cp(b, b).start()
        for b in range(2):
            didx_cp(b, b).start()
        for b in range(2):
            sidx_cp(b, b).wait()
            g_sp(b).start()
            g_t(b).start()

        def body(i, carry):
            for t in range(4):
                j = 4 * i + t

                @pl.when(j < _NB)
                def _():
                    b = t
                    b2 = (t + 2) % 4
                    g_sp(b).wait()
                    g_t(b).wait()

                    @pl.when(j + 4 < _NB)
                    def _():
                        sidx_cp(j + 4, b).start()

                    didx_cp(j, b).wait()
                    s_sp(b).start(add=True)
                    s_t(b).start(add=True)

                    @pl.when(j >= 2)
                    def _():
                        s_sp(b2).wait()
                        s_t(b2).wait()

                    @pl.when(j + 2 < _NB)
                    def _():
                        didx_cp(j + 2, b2).start()
                        sidx_cp(j + 2, b2).wait()
                        g_sp(b2).start()
                        g_t(b2).start()

            return carry

        lax.fori_loop(0, (_NB + 3) // 4, body, 0)
        for j in (_NB - 2, _NB - 1):
            b = j % 4
            s_sp(b).wait()
            s_t(b).wait()
        plsc.subcore_barrier()
        pltpu.sync_copy(accs.at[pl.ds(s * _RPS, _RPS)],
                        osp_hbm.at[c, pl.ds(s * _RPS, _RPS)])
        pltpu.sync_copy(acct.at[pl.ds(s * _RPS, _RPS)],
                        ot_hbm.at[c, pl.ds(s * _RPS, _RPS)])

    return k(h_sp, h_t, ei3, z_sp, z_t)


def kernel(node_feat, edge_index, W1, b1, s1, W2, b2, s2, cls, bias_dec):
    f32 = jnp.float32
    b1r = jnp.reshape(b1, (1, 129))
    b2r = jnp.reshape(b2, (1, 129))
    bdr = jnp.reshape(bias_dec, (1, 7))
    s1a = jnp.reshape(s1, (1, 1)).astype(f32)
    s2a = jnp.reshape(s2, (1, 1)).astype(f32)
    ei3 = jnp.reshape(edge_index, (2, _NW * _NB, _EB))
    z_sp = jnp.zeros((_RPS, _PS), f32)
    z_t = jnp.zeros((_RPS, _PT), f32)

    h1s, h1t = _tc_expmap_linear(node_feat, W1, b1r, s1a)
    a1s, a1t = _sc_segment_sum(h1s, h1t, ei3, z_sp, z_t)
    h2s, h2t = _tc_agg_linear(a1s, a1t, W2, b2r, s2a)
    a2s, a2t = _sc_segment_sum(h2s, h2t, ei3, z_sp, z_t)
    return _tc_agg_decode(a2s, a2t, cls, bdr)


# speedup vs baseline: 10.7661x; 1.0127x over previous
import functools

import jax
import jax.numpy as jnp
from jax import lax
from jax.experimental import pallas as pl
from jax.experimental.pallas import tpu as pltpu
from jax.experimental.pallas import tpu_sc as plsc

_N = 10000
_E = 320000
_PS = 128
_PT = 8
_EB = 80
_NC = 2
_NS = 16
_NW = _NC * _NS
_NB = _E // (_NW * _EB)
_NA = 10112
_RPS = _NA // _NS
_BN = 2000


def _lorentz_tail(y, s_raw):
    sfac = jnp.minimum(jnp.exp(s_raw), 10.0)
    t = sfac / (1.0 + jnp.exp(-y[:, 0:1])) + 1.5
    sq = jnp.sum(y * y, axis=1, keepdims=True) - y[:, 0:1] * y[:, 0:1]
    sq = jnp.maximum(sq, 1e-8)
    fac = jnp.sqrt(jnp.maximum((t * t - 1.0) / sq, 1e-8))
    space = y[:, 1:129] * fac
    col = lax.broadcasted_iota(jnp.int32, (y.shape[0], _PT), 1)
    time8 = jnp.where(col == 0, t, 0.0)
    return space, time8


def _nt_dot(x, w):
    return lax.dot_general(x, w, (((1,), (1,)), ((), ())),
                           preferred_element_type=jnp.float32)


def _expmap_linear_body(x_ref, w_ref, b_ref, s_ref, osp_ref, ot_ref):
    x = x_ref[...]
    nrm = jnp.maximum(jnp.sqrt(jnp.sum(x * x, axis=1, keepdims=True)), 1e-8)
    e = jnp.exp(nrm)
    ei = 1.0 / e
    time = 0.5 * (e + ei)
    coef = (0.5 * (e - ei)) / nrm
    w = w_ref[...]
    y = _nt_dot(coef * x, w[:, 1:]) + time * w[:, 0] + b_ref[...]
    osp_ref[...], ot_ref[...] = _lorentz_tail(y, s_ref[0, 0])


def _agg_linear_body(p0_ref, p1_ref, q0_ref, q1_ref, w_ref, b_ref, s_ref,
                     osp_ref, ot_ref):
    sup = p0_ref[0] + p1_ref[0]
    t0 = (q0_ref[0] + q1_ref[0])[:, 0:1]
    inner = jnp.sum(sup * sup, axis=1, keepdims=True) - t0 * t0
    denom = jnp.sqrt(jnp.maximum(jnp.abs(inner), 1e-8))
    xr = jnp.maximum(sup / denom, 0.0)
    xt = jnp.maximum(t0 / denom, 0.0)
    w = w_ref[...]
    y = _nt_dot(xr, w[:, 1:]) + xt * w[:, 0] + b_ref[...]
    osp_ref[...], ot_ref[...] = _lorentz_tail(y, s_ref[0, 0])


def _agg_decode_body(p0_ref, p1_ref, q0_ref, q1_ref, cls_ref, b_ref, o_ref):
    sup = p0_ref[0] + p1_ref[0]
    t0 = (q0_ref[0] + q1_ref[0])[:, 0:1]
    inner = jnp.sum(sup * sup, axis=1, keepdims=True) - t0 * t0
    denom = jnp.sqrt(jnp.maximum(jnp.abs(inner), 1e-8))
    h = sup / denom
    ht = t0 / denom
    cw = cls_ref[...]
    y = _nt_dot(h, cw[:, 1:]) - ht * cw[:, 0]
    o_ref[...] = 2.0 + 2.0 * y + b_ref[...]


def _tc_expmap_linear(x, w, b, s):
    return pl.pallas_call(
        _expmap_linear_body,
        grid=(_N // _BN,),
        in_specs=[
            pl.BlockSpec((_BN, _PS), lambda i: (i, 0)),
            pl.BlockSpec((129, 129), lambda i: (0, 0)),
            pl.BlockSpec((1, 129), lambda i: (0, 0)),
            pl.BlockSpec((1, 1), lambda i: (0, 0)),
        ],
        out_specs=[pl.BlockSpec((_BN, _PS), lambda i: (i, 0)),
                   pl.BlockSpec((_BN, _PT), lambda i: (i, 0))],
        out_shape=[jax.ShapeDtypeStruct((_N, _PS), jnp.float32),
                   jax.ShapeDtypeStruct((_N, _PT), jnp.float32)],
    )(x, w, b, s)


def _tc_agg_linear(psp, pt, w, b, s):
    return pl.pallas_call(
        _agg_linear_body,
        grid=(_N // _BN,),
        in_specs=[
            pl.BlockSpec((1, _BN, _PS), lambda i: (0, i, 0)),
            pl.BlockSpec((1, _BN, _PS), lambda i: (1, i, 0)),
            pl.BlockSpec((1, _BN, _PT), lambda i: (0, i, 0)),
            pl.BlockSpec((1, _BN, _PT), lambda i: (1, i, 0)),
            pl.BlockSpec((129, 129), lambda i: (0, 0)),
            pl.BlockSpec((1, 129), lambda i: (0, 0)),
            pl.BlockSpec((1, 1), lambda i: (0, 0)),
        ],
        out_specs=[pl.BlockSpec((_BN, _PS), lambda i: (i, 0)),
                   pl.BlockSpec((_BN, _PT), lambda i: (i, 0))],
        out_shape=[jax.ShapeDtypeStruct((_N, _PS), jnp.float32),
                   jax.ShapeDtypeStruct((_N, _PT), jnp.float32)],
    )(psp, psp, pt, pt, w, b, s)


def _tc_agg_decode(psp, pt, clsw, bd):
    return pl.pallas_call(
        _agg_decode_body,
        grid=(_N // _BN,),
        in_specs=[
            pl.BlockSpec((1, _BN, _PS), lambda i: (0, i, 0)),
            pl.BlockSpec((1, _BN, _PS), lambda i: (1, i, 0)),
            pl.BlockSpec((1, _BN, _PT), lambda i: (0, i, 0)),
            pl.BlockSpec((1, _BN, _PT), lambda i: (1, i, 0)),
            pl.BlockSpec((7, 129), lambda i: (0, 0)),
            pl.BlockSpec((1, 7), lambda i: (0, 0)),
        ],
        out_specs=pl.BlockSpec((_BN, 7), lambda i: (i, 0)),
        out_shape=jax.ShapeDtypeStruct((_N, 7), jnp.float32),
    )(psp, psp, pt, pt, clsw, bd)


def _sc_segment_sum(h_sp, h_t, ei3, z_sp, z_t):
    mesh = plsc.VectorSubcoreMesh(core_axis_name="c", subcore_axis_name="s")

    @functools.partial(
        pl.kernel,
        mesh=mesh,
        compiler_params=pltpu.CompilerParams(use_tc_tiling_on_sc=False),
        out_type=[jax.ShapeDtypeStruct((_NC, _NA, _PS), jnp.float32),
                  jax.ShapeDtypeStruct((_NC, _NA, _PT), jnp.float32)],
        scratch_types=[
            pltpu.VMEM((4, _EB), jnp.int32),
            pltpu.VMEM((4, _EB), jnp.int32),
            pltpu.VMEM((4, _EB, _PS), jnp.float32),
            pltpu.VMEM((4, _EB, _PT), jnp.float32),
            pltpu.VMEM_SHARED((_NA, _PS), jnp.float32),
            pltpu.VMEM_SHARED((_NA, _PT), jnp.float32),
            [pltpu.SemaphoreType.DMA] * 4,
            [pltpu.SemaphoreType.DMA] * 4,
            [pltpu.SemaphoreType.DMA] * 4,
            [pltpu.SemaphoreType.DMA] * 4,
        ],
    )
    def k(hsp_hbm, ht_hbm, ei_hbm, zsp_hbm, zt_hbm, osp_hbm, ot_hbm,
          src_v, dst_v, rows_v, trow_v, accs, acct,
          gsem, ssem, isem, dsem):
        c = lax.axis_index("c")
        s = lax.axis_index("s")
        wid = s * _NC + c
        pltpu.sync_copy(zsp_hbm, accs.at[pl.ds(s * _RPS, _RPS)])
        pltpu.sync_copy(zt_hbm, acct.at[pl.ds(s * _RPS, _RPS)])
        plsc.subcore_barrier()

        def sidx_cp(j, b):
            return pltpu.make_async_copy(
                ei_hbm.at[0, pl.ds(wid * _NB + j, 1)],
                src_v.at[pl.ds(b, 1)], isem[b])

        def didx_cp(j, b):
            return pltpu.make_async_copy(
                ei_hbm.at[1, pl.ds(wid * _NB + j, 1)],
                dst_v.at[pl.ds(b, 1)], dsem[b])

        def g_sp(b):
            return pltpu.make_async_copy(hsp_hbm.at[src_v.at[b]],
                                         rows_v.at[b], gsem[b])

        def g_t(b):
            return pltpu.make_async_copy(ht_hbm.at[src_v.at[b]],
                                         trow_v.at[b], gsem[b])

        def s_sp(b):
            return pltpu.make_async_copy(rows_v.at[b], accs.at[dst_v.at[b]],
                                         ssem[b])

        def s_t(b):
            return pltpu.make_async_copy(trow_v.at[b], acct.at[dst_v.at[b]],
                                         ssem[b])

        for b in range(4):
            sidx_cp(b, b).start()
        for b in range(2):
            didx_cp(b, b).start()
        for b in range(2):
            sidx_cp(b, b).wait()
            g_sp(b).start()
            g_t(b).start()

        def body(i, carry):
            for t in range(4):
                j = 4 * i + t

                @pl.when(j < _NB)
                def _():
                    b = t
                    b2 = (t + 2) % 4
                    g_sp(b).wait()
                    g_t(b).wait()

                    @pl.when(j + 4 < _NB)
                    def _():
                        sidx_cp(j + 4, b).start()

                    didx_cp(j, b).wait()
                    s_sp(b).start(add=True)
                    s_t(b).start(add=True)

                    @pl.when(j >= 2)
                    def _():
                        s_sp(b2).wait()
                        s_t(b2).wait()

                    @pl.when(j + 2 < _NB)
                    def _():
                        didx_cp(j + 2, b2).start()
                        sidx_cp(j + 2, b2).wait()
                        g_sp(b2).start()
                        g_t(b2).start()

            return carry

        lax.fori_loop(0, (_NB + 3) // 4, body, 0)
        for j in (_NB - 2, _NB - 1):
            b = j % 4
            s_sp(b).wait()
            s_t(b).wait()
        plsc.subcore_barrier()
        pltpu.sync_copy(accs.at[pl.ds(s * _RPS, _RPS)],
                        osp_hbm.at[c, pl.ds(s * _RPS, _RPS)])
        pltpu.sync_copy(acct.at[pl.ds(s * _RPS, _RPS)],
                        ot_hbm.at[c, pl.ds(s * _RPS, _RPS)])

    return k(h_sp, h_t, ei3, z_sp, z_t)


def kernel(node_feat, edge_index, W1, b1, s1, W2, b2, s2, cls, bias_dec):
    f32 = jnp.float32
    b1r = jnp.reshape(b1, (1, 129))
    b2r = jnp.reshape(b2, (1, 129))
    bdr = jnp.reshape(bias_dec, (1, 7))
    s1a = jnp.reshape(s1, (1, 1)).astype(f32)
    s2a = jnp.reshape(s2, (1, 1)).astype(f32)
    ei3 = jnp.reshape(edge_index, (2, _NW * _NB, _EB))
    z_sp = jnp.zeros((_RPS, _PS), f32)
    z_t = jnp.zeros((_RPS, _PT), f32)

    h1s, h1t = _tc_expmap_linear(node_feat, W1, b1r, s1a)
    a1s, a1t = _sc_segment_sum(h1s, h1t, ei3, z_sp, z_t)
    h2s, h2t = _tc_agg_linear(a1s, a1t, W2, b2r, s2a)
    a2s, a2t = _sc_segment_sum(h2s, h2t, ei3, z_sp, z_t)
    return _tc_agg_decode(a2s, a2t, cls, bdr)
